# baseline (speedup 1.0000x reference)
import jax
import jax.numpy as jnp
from jax import lax
from jax.experimental import pallas as pl
from jax.experimental.pallas import tpu as pltpu
from jax.experimental.pallas import tpu_sc as plsc

VOCAB = 1000000
EMBED = 64
WIDE = 128
BATCH = 16384
CTX = 50

NC = 2
NS = 16
NW = NC * NS

CBLK = 128
NFULL = VOCAB // CBLK
NFULL_PER_W = NFULL // NW
NBUF = 4
TAIL_COLS = VOCAB - NFULL * CBLK

ROWS_PER_DMA = 4
CHUNK = ROWS_PER_DMA * CTX
RPW = BATCH // NW
CPW = RPW // ROWS_PER_DMA
NGRP = CPW
NLANE = EMBED // 16
INV_CTX = 1.0 / CTX


def _transpose_body(tt_hbm, tail_hbm, t2_hbm, in_v, out_v,
                    is0, is1, is2, is3, os0, os1, os2, os3):
    wid = lax.axis_index("s") * NC + lax.axis_index("c")
    iota = lax.broadcasted_iota(jnp.int32, (16,), 0)

    isems = (is0, is1, is2, is3)
    osems = (os0, os1, os2, os3)

    def blk(i):
        return i * NW + wid

    def issue_in(i, s):
        pltpu.make_async_copy(
            tt_hbm.at[:, pl.ds(blk(i) * CBLK, CBLK)],
            in_v.at[s], isems[s]).start()

    def wait_in(i, s):
        pltpu.make_async_copy(
            tt_hbm.at[:, pl.ds(blk(i) * CBLK, CBLK)],
            in_v.at[s], isems[s]).wait()

    def issue_out(i, s):
        pltpu.make_async_copy(
            out_v.at[s],
            t2_hbm.at[pl.ds(blk(i) * (CBLK // 2), CBLK // 2)], osems[s]).start()

    def wait_out(i, s):
        pltpu.make_async_copy(
            out_v.at[s],
            t2_hbm.at[pl.ds(blk(i) * (CBLK // 2), CBLK // 2)], osems[s]).wait()

    cvecs = [iota + 16 * cg for cg in range(EMBED // 16)]

    def transpose_block(s):
        inb = in_v.at[s]

        def pair_body(m, carry):
            r0 = jnp.full((16,), 2 * m, jnp.int32)
            r1 = jnp.full((16,), 2 * m + 1, jnp.int32)
            for cg in range(EMBED // 16):
                v0 = plsc.load_gather(inb, [cvecs[cg], r0])
                out_v[s, m, pl.ds(16 * cg, 16)] = v0
            for cg in range(EMBED // 16):
                v1 = plsc.load_gather(inb, [cvecs[cg], r1])
                out_v[s, m, pl.ds(EMBED + 16 * cg, 16)] = v1
            return carry

        lax.fori_loop(0, CBLK // 2, pair_body, 0, unroll=4)

    for s in range(NBUF):
        issue_in(s, s)

    NSUPER = NFULL_PER_W // NBUF

    def super_body(si, carry):
        for s in range(NBUF):
            i = si * NBUF + s
            wait_in(i, s)

            @pl.when(si >= 1)
            def _():
                wait_out(i - NBUF, s)
            transpose_block(s)
            issue_out(i, s)

            @pl.when(si < NSUPER - 1)
            def _():
                issue_in(i + NBUF, s)
        return carry

    lax.fori_loop(0, NSUPER, super_body, 0, unroll=False)
    for s in range(NBUF):
        wait_out((NSUPER - 1) * NBUF + s, s)

    @pl.when(wid < NFULL - NW * NFULL_PER_W)
    def _extra():
        b = NFULL_PER_W * NW + wid
        pltpu.sync_copy(tt_hbm.at[:, pl.ds(b * CBLK, CBLK)], in_v.at[0])
        transpose_block(0)
        pltpu.sync_copy(out_v.at[0], t2_hbm.at[pl.ds(b * (CBLK // 2), CBLK // 2)])

    @pl.when(wid == 0)
    def _tail():
        pltpu.sync_copy(tail_hbm, out_v.at[0, pl.ds(0, TAIL_COLS // 2)])
        pltpu.sync_copy(out_v.at[0, pl.ds(0, TAIL_COLS // 2)],
                        t2_hbm.at[pl.ds(NFULL * (CBLK // 2), TAIL_COLS // 2)])


def _cbow_body(pidx_hbm, par_hbm, table_hbm, out_hbm,
               pidx_v, par_v, buf_v, out_v, sem0, sem1):
    wid = lax.axis_index("s") * NC + lax.axis_index("c")

    pltpu.sync_copy(pidx_hbm.at[pl.ds(wid * CPW * CHUNK, CPW * CHUNK)], pidx_v)
    pltpu.sync_copy(par_hbm.at[pl.ds(wid * RPW * 2, RPW * 2)], par_v)

    sems = (sem0, sem1)
    iota = lax.broadcasted_iota(jnp.int32, (16,), 0)
    base_q = [iota + 16 * q for q in range(NLANE)]

    def issue(c, parity):
        pltpu.make_async_copy(
            table_hbm.at[pidx_v.at[pl.ds(c * CHUNK, CHUNK)]],
            buf_v.at[parity, 0],
            sems[parity],
        ).start()

    def drain(c, parity):
        pltpu.make_async_copy(
            table_hbm.at[pidx_v.at[pl.ds(c * CHUNK, CHUNK)]],
            buf_v.at[parity, 0],
            sems[parity],
        ).wait()

    def reduce_chunk(g, parity):
        buf = buf_v.at[parity, 0]

        def row_body(rr, carry):
            orow = g * ROWS_PER_DMA + rr
            w0 = plsc.load_gather(par_v, [jnp.full((16,), 2 * orow, jnp.int32)])
            w1 = plsc.load_gather(par_v, [jnp.full((16,), 2 * orow + 1, jnp.int32)])
            acc = [None] * NLANE
            for j in range(CTX):
                w, sh = (w0, j) if j < 32 else (w1, j - 32)
                poff = lax.shift_left(
                    lax.bitwise_and(lax.shift_right_logical(w, sh), 1), 6)
                rvec = jnp.full((16,), rr * CTX + j, jnp.int32)
                for q in range(NLANE):
                    g_q = plsc.load_gather(buf, [rvec, poff + base_q[q]])
                    acc[q] = g_q if acc[q] is None else acc[q] + g_q
            for q in range(NLANE):
                out_v[pl.ds(orow * EMBED + 16 * q, 16)] = acc[q] * INV_CTX
            return carry

        lax.fori_loop(0, ROWS_PER_DMA, row_body, 0, unroll=False)

    issue(0, 0)

    def group_body(g, carry):
        parity = lax.rem(g, 2)

        @pl.when(g + 1 < NGRP)
        def _issue_next():
            nparity = lax.rem(g + 1, 2)

            @pl.when(nparity == 0)
            def _():
                issue(g + 1, 0)

            @pl.when(nparity == 1)
            def _():
                issue(g + 1, 1)

        @pl.when(parity == 0)
        def _p0():
            drain(g, 0)
            reduce_chunk(g, 0)

        @pl.when(parity == 1)
        def _p1():
            drain(g, 1)
            reduce_chunk(g, 1)

        return carry

    lax.fori_loop(0, NGRP, group_body, 0, unroll=False)

    pltpu.sync_copy(out_v, out_hbm.at[pl.ds(wid * RPW * EMBED, RPW * EMBED)])


@jax.jit
def _cbow(pidx, parw, tableT, tail2):
    mesh = plsc.VectorSubcoreMesh(core_axis_name="c", subcore_axis_name="s")
    params = pltpu.CompilerParams(
        use_tc_tiling_on_sc=True, needs_layout_passes=False)

    t2 = pl.kernel(
        _transpose_body,
        out_type=jax.ShapeDtypeStruct((VOCAB // 2, WIDE), jnp.float32),
        mesh=mesh,
        scratch_types=[
            pltpu.VMEM((NBUF, EMBED, CBLK), jnp.float32),
            pltpu.VMEM((NBUF, EMBED, WIDE), jnp.float32),
        ] + [pltpu.SemaphoreType.DMA] * 8,
        compiler_params=params,
    )(tableT, tail2)

    out = pl.kernel(
        _cbow_body,
        out_type=jax.ShapeDtypeStruct((BATCH * EMBED,), jnp.float32),
        mesh=mesh,
        scratch_types=[
            pltpu.VMEM((CPW * CHUNK,), jnp.int32),
            pltpu.VMEM((RPW * 2,), jnp.int32),
            pltpu.VMEM((2, 1, CHUNK, WIDE), jnp.float32),
            pltpu.VMEM((RPW * EMBED,), jnp.float32),
            pltpu.SemaphoreType.DMA,
            pltpu.SemaphoreType.DMA,
        ],
        compiler_params=params,
    )(pidx, parw, t2)
    return out


def kernel(inputs, table):
    idx = inputs.astype(jnp.int32)
    pidx = lax.shift_right_logical(idx, 1).reshape(-1)
    par = lax.bitwise_and(idx, 1)
    sh = jnp.arange(32, dtype=jnp.int32)
    w0 = lax.shift_left(par[:, :32], sh[None, :]).sum(axis=1)
    w1 = lax.shift_left(par[:, 32:], sh[None, :CTX - 32]).sum(axis=1)
    parw = jnp.stack([w0, w1], axis=1).reshape(-1)
    tail2 = table[VOCAB - TAIL_COLS:].reshape(TAIL_COLS // 2, WIDE)
    return _cbow(pidx, parw, table.T, tail2).reshape(BATCH, EMBED)


# speedup vs baseline: 1.8300x; 1.8300x over previous
import jax
import jax.numpy as jnp
from jax import lax
from jax.experimental import pallas as pl
from jax.experimental.pallas import tpu as pltpu
from jax.experimental.pallas import tpu_sc as plsc

VOCAB = 1000000
EMBED = 64
WIDE = 128
BATCH = 16384
CTX = 50

NC = 2
NS = 16
NW = NC * NS

TP = 512
SPLIT = TP * 980
TGRID = SPLIT // TP

ROWS_PER_DMA = 4
CHUNK = ROWS_PER_DMA * CTX
RPW = BATCH // NW
CPW = RPW // ROWS_PER_DMA
NGRP = CPW
NLANE = EMBED // 16
INV_CTX = 1.0 / CTX


def _tr_body(a_ref, b_ref, o_ref):
    o_ref[:, 0:EMBED] = jnp.transpose(a_ref[...], (1, 0))
    o_ref[:, EMBED:WIDE] = jnp.transpose(b_ref[...], (1, 0))


def _cbow_body(pidx_hbm, par_hbm, table_hbm, out_hbm,
               pidx_v, par_v, buf_v, out_v, sem0, sem1):
    wid = lax.axis_index("s") * NC + lax.axis_index("c")

    pltpu.sync_copy(pidx_hbm.at[pl.ds(wid * CPW * CHUNK, CPW * CHUNK)], pidx_v)
    pltpu.sync_copy(par_hbm.at[pl.ds(wid * RPW * 2, RPW * 2)], par_v)

    sems = (sem0, sem1)
    iota = lax.broadcasted_iota(jnp.int32, (16,), 0)
    base_q = [iota + 16 * q for q in range(NLANE)]

    def issue(c, parity):
        pltpu.make_async_copy(
            table_hbm.at[pidx_v.at[pl.ds(c * CHUNK, CHUNK)]],
            buf_v.at[parity, 0],
            sems[parity],
        ).start()

    def drain(c, parity):
        pltpu.make_async_copy(
            table_hbm.at[pidx_v.at[pl.ds(c * CHUNK, CHUNK)]],
            buf_v.at[parity, 0],
            sems[parity],
        ).wait()

    def reduce_chunk(g, parity):
        buf = buf_v.at[parity, 0]

        def row_body(rr, carry):
            orow = g * ROWS_PER_DMA + rr
            w0 = plsc.load_gather(par_v, [jnp.full((16,), 2 * orow, jnp.int32)])
            w1 = plsc.load_gather(par_v, [jnp.full((16,), 2 * orow + 1, jnp.int32)])
            acc = [None] * NLANE
            for j in range(CTX):
                w, sh = (w0, j) if j < 32 else (w1, j - 32)
                poff = lax.shift_left(
                    lax.bitwise_and(lax.shift_right_logical(w, sh), 1), 6)
                rvec = jnp.full((16,), rr * CTX + j, jnp.int32)
                for q in range(NLANE):
                    g_q = plsc.load_gather(buf, [rvec, poff + base_q[q]])
                    acc[q] = g_q if acc[q] is None else acc[q] + g_q
            for q in range(NLANE):
                out_v[pl.ds(orow * EMBED + 16 * q, 16)] = acc[q] * INV_CTX
            return carry

        lax.fori_loop(0, ROWS_PER_DMA, row_body, 0, unroll=False)

    issue(0, 0)

    def group_body(g, carry):
        parity = lax.rem(g, 2)

        @pl.when(g + 1 < NGRP)
        def _issue_next():
            nparity = lax.rem(g + 1, 2)

            @pl.when(nparity == 0)
            def _():
                issue(g + 1, 0)

            @pl.when(nparity == 1)
            def _():
                issue(g + 1, 1)

        @pl.when(parity == 0)
        def _p0():
            drain(g, 0)
            reduce_chunk(g, 0)

        @pl.when(parity == 1)
        def _p1():
            drain(g, 1)
            reduce_chunk(g, 1)

        return carry

    lax.fori_loop(0, NGRP, group_body, 0, unroll=False)

    pltpu.sync_copy(out_v, out_hbm.at[pl.ds(wid * RPW * EMBED, RPW * EMBED)])


@jax.jit
def _cbow(pidx, parw, tableT):
    mesh = plsc.VectorSubcoreMesh(core_axis_name="c", subcore_axis_name="s")
    params = pltpu.CompilerParams(
        use_tc_tiling_on_sc=True, needs_layout_passes=False)

    t2 = pl.pallas_call(
        _tr_body,
        grid=(TGRID,),
        in_specs=[
            pl.BlockSpec((EMBED, TP), lambda i: (0, i)),
            pl.BlockSpec((EMBED, TP),
                         lambda i: (0, jnp.minimum(i + TGRID, VOCAB // TP))),
        ],
        out_specs=pl.BlockSpec((TP, WIDE), lambda i: (i, 0)),
        out_shape=jax.ShapeDtypeStruct((SPLIT, WIDE), jnp.float32),
    )(tableT, tableT)

    out = pl.kernel(
        _cbow_body,
        out_type=jax.ShapeDtypeStruct((BATCH * EMBED,), jnp.float32),
        mesh=mesh,
        scratch_types=[
            pltpu.VMEM((CPW * CHUNK,), jnp.int32),
            pltpu.VMEM((RPW * 2,), jnp.int32),
            pltpu.VMEM((2, 1, CHUNK, WIDE), jnp.float32),
            pltpu.VMEM((RPW * EMBED,), jnp.float32),
            pltpu.SemaphoreType.DMA,
            pltpu.SemaphoreType.DMA,
        ],
        compiler_params=params,
    )(pidx, parw, t2)
    return out


def kernel(inputs, table):
    idx = inputs.astype(jnp.int32)
    par = (idx >= SPLIT).astype(jnp.int32)
    pidx = (idx - par * SPLIT).reshape(-1)
    sh = jnp.arange(32, dtype=jnp.int32)
    w0 = lax.shift_left(par[:, :32], sh[None, :]).sum(axis=1)
    w1 = lax.shift_left(par[:, 32:], sh[None, :CTX - 32]).sum(axis=1)
    parw = jnp.stack([w0, w1], axis=1).reshape(-1)
    return _cbow(pidx, parw, table.T).reshape(BATCH, EMBED)


# speedup vs baseline: 2.3201x; 1.2678x over previous
import jax
import jax.numpy as jnp
from jax import lax
from jax.experimental import pallas as pl
from jax.experimental.pallas import tpu as pltpu
from jax.experimental.pallas import tpu_sc as plsc

VOCAB = 1000000
EMBED = 64
EMBED_PAD = 128
BATCH = 16384
CTX = 50

NC = 2
NS = 16
NW = NC * NS

ROWS_PER_DMA = 8
CHUNK = ROWS_PER_DMA * CTX
CHUNK_PAD = 400
RPW = BATCH // NW
CPW = RPW // ROWS_PER_DMA
GSIZE = 1
NGRP = CPW // GSIZE
GROWS = GSIZE * ROWS_PER_DMA
NLANE = EMBED // 16
INV_CTX = 1.0 / CTX


def _cbow_body(idx_hbm, table_hbm, out_hbm, idx_v, buf_v, out_v, sem0, sem1):
    wid = lax.axis_index("s") * NC + lax.axis_index("c")
    base_chunk = wid * CPW

    pltpu.sync_copy(idx_hbm.at[pl.ds(base_chunk * CHUNK_PAD, CPW * CHUNK_PAD)],
                    idx_v)

    sems = (sem0, sem1)

    def issue_group(g, parity):
        for k in range(GSIZE):
            c = g * GSIZE + k
            pltpu.make_async_copy(
                table_hbm.at[idx_v.at[pl.ds(c * CHUNK_PAD, CHUNK_PAD)]],
                buf_v.at[parity, k],
                sems[parity],
            ).start()

    def drain_group(g, parity):
        for k in range(GSIZE):
            c = g * GSIZE + k
            pltpu.make_async_copy(
                table_hbm.at[idx_v.at[pl.ds(c * CHUNK_PAD, CHUNK_PAD)]],
                buf_v.at[parity, k],
                sems[parity],
            ).wait()

    def reduce_group(g, parity):
        def row_body(rr, carry):
            c = rr // ROWS_PER_DMA
            r = rr % ROWS_PER_DMA
            j0 = r * CTX
            acc = [buf_v[parity, c, j0, pl.ds(16 * q, 16)] for q in range(NLANE)]
            for j in range(1, CTX):
                for q in range(NLANE):
                    acc[q] += buf_v[parity, c, j0 + j, pl.ds(16 * q, 16)]
            orow = g * GROWS + rr
            for q in range(NLANE):
                out_v[orow, pl.ds(16 * q, 16)] = acc[q] * INV_CTX
            return carry
        lax.fori_loop(0, GROWS, row_body, 0, unroll=False)

    issue_group(0, 0)

    def group_body(g, carry):
        parity = lax.rem(g, 2)

        @pl.when(g + 1 < NGRP)
        def _issue_next():
            nparity = lax.rem(g + 1, 2)

            @pl.when(nparity == 0)
            def _():
                issue_group(g + 1, 0)

            @pl.when(nparity == 1)
            def _():
                issue_group(g + 1, 1)

        @pl.when(parity == 0)
        def _p0():
            drain_group(g, 0)
            reduce_group(g, 0)

        @pl.when(parity == 1)
        def _p1():
            drain_group(g, 1)
            reduce_group(g, 1)

        return carry

    lax.fori_loop(0, NGRP, group_body, 0, unroll=False)

    pltpu.sync_copy(out_v, out_hbm.at[pl.ds(wid * RPW, RPW)])


@jax.jit
def _cbow(idx_padded, table):
    mesh = plsc.VectorSubcoreMesh(core_axis_name="c", subcore_axis_name="s")
    f = pl.kernel(
        _cbow_body,
        out_type=jax.ShapeDtypeStruct((BATCH, EMBED), jnp.float32),
        mesh=mesh,
        scratch_types=[
            pltpu.VMEM((CPW * CHUNK_PAD,), jnp.int32),
            pltpu.VMEM((2, GSIZE, CHUNK_PAD, EMBED), jnp.float32),
            pltpu.VMEM((RPW, EMBED), jnp.float32),
            pltpu.SemaphoreType.DMA,
            pltpu.SemaphoreType.DMA,
        ],
        compiler_params=pltpu.CompilerParams(use_tc_tiling_on_sc=False),
    )
    return f(idx_padded, table)


def kernel(inputs, table):
    idx = inputs.astype(jnp.int32).reshape(BATCH // ROWS_PER_DMA, CHUNK)
    idx = jnp.pad(idx, ((0, 0), (0, CHUNK_PAD - CHUNK))).reshape(-1)
    return _cbow(idx, table)
